# baseline (speedup 1.0000x reference)
import functools
import math

import jax
import jax.numpy as jnp
from jax import lax
from jax.experimental import pallas as pl
from jax.experimental.pallas import tpu as pltpu
from jax.experimental.pallas import tpu_sc as plsc

_NUM_CORES = 2
_NUM_SUBCORES = 16
_NUM_WORKERS = _NUM_CORES * _NUM_SUBCORES
_LANES = 16


def _emb_body(batch, seq, d_emb, scale,
              x_hbm, table_hbm, pos_hbm, out_hbm,
              idx_v, rows_v, pos_v, si, sp, sg0, sg1, sg2, sg3, sw):
    wid = lax.axis_index("s") * _NUM_CORES + lax.axis_index("c")
    seq_per_w = seq // _NUM_WORKERS
    seq_base = wid * seq_per_w
    n_col_chunks = d_emb // _LANES
    gather_sems = (sg0, sg1, sg2, sg3)

    idx_cps = [
        pltpu.async_copy(x_hbm.at[c, wid], idx_v.at[c], si) for c in range(batch)
    ]
    pos_cp = pltpu.async_copy(pos_hbm.at[pl.ds(seq_base, seq_per_w)], pos_v, sp)
    for cp in idx_cps:
        cp.wait()
    gathers = [
        pltpu.async_copy(table_hbm.at[idx_v.at[c]], rows_v.at[c], gather_sems[c])
        for c in range(batch)
    ]
    pos_cp.wait()

    writebacks = []
    group = 16
    for pair in range(batch // 2):
        c0, c1 = 2 * pair, 2 * pair + 1
        gathers[c0].wait()
        gathers[c1].wait()

        for g in range(seq_per_w // group):
            g_base = g * group

            def row(r, carry, _c0=c0, _c1=c1):
                slices = [pl.ds(cc * _LANES, _LANES) for cc in range(n_col_chunks)]
                pos_regs = [pos_v[r, sl] for sl in slices]
                for c in (_c0, _c1):
                    for cc, sl in enumerate(slices):
                        rows_v[c, r, sl] = rows_v[c, r, sl] * scale + pos_regs[cc]
                return carry

            lax.fori_loop(g_base, g_base + group, row, 0)
            for c in (c0, c1):
                writebacks.append(
                    pltpu.async_copy(
                        rows_v.at[c, pl.ds(g_base, group)],
                        out_hbm.at[pl.ds(c * seq + seq_base + g_base, group)], sw))
    for wb in writebacks:
        wb.wait()


def kernel(x, table, pos_encoding):
    batch, seq = x.shape
    _, d_emb = table.shape
    n = batch * seq
    seq_per_w = seq // _NUM_WORKERS
    scale = math.sqrt(d_emb)

    mesh = plsc.VectorSubcoreMesh(core_axis_name="c", subcore_axis_name="s")
    body = functools.partial(_emb_body, batch, seq, d_emb, scale)
    run = pl.kernel(
        body,
        mesh=mesh,
        out_type=jax.ShapeDtypeStruct((n, d_emb), jnp.float32),
        scratch_types=[
            pltpu.VMEM((batch, seq_per_w), jnp.int32),
            pltpu.VMEM((batch, seq_per_w, d_emb), jnp.float32),
            pltpu.VMEM((seq_per_w, d_emb), jnp.float32),
            pltpu.SemaphoreType.DMA,
            pltpu.SemaphoreType.DMA,
            pltpu.SemaphoreType.DMA,
            pltpu.SemaphoreType.DMA,
            pltpu.SemaphoreType.DMA,
            pltpu.SemaphoreType.DMA,
            pltpu.SemaphoreType.DMA,
        ],
    )
    out = run(x.reshape(batch, _NUM_WORKERS, seq_per_w), table, pos_encoding[:seq])
    return out.reshape(batch, seq, d_emb)


# speedup vs baseline: 1.0203x; 1.0203x over previous
import functools
import math

import jax
import jax.numpy as jnp
from jax import lax
from jax.experimental import pallas as pl
from jax.experimental.pallas import tpu as pltpu
from jax.experimental.pallas import tpu_sc as plsc

_NUM_CORES = 2
_NUM_SUBCORES = 16
_NUM_WORKERS = _NUM_CORES * _NUM_SUBCORES
_LANES = 16


def _emb_body(batch, seq, d_emb, scale,
              x_hbm, table_hbm, pos_hbm, out_hbm,
              idx_v, rows_v, pos_v, si, sp, sg0, sg1, sg2, sg3, sw):
    wid = lax.axis_index("s") * _NUM_CORES + lax.axis_index("c")
    seq_per_w = seq // _NUM_WORKERS
    seq_base = wid * seq_per_w
    n_col_chunks = d_emb // _LANES
    gather_sems = (sg0, sg1, sg2, sg3)

    idx_cps = [
        pltpu.async_copy(x_hbm.at[c, pl.ds(seq_base, seq_per_w)], idx_v.at[c], si)
        for c in range(batch)
    ]
    pos_cp = pltpu.async_copy(pos_hbm.at[pl.ds(seq_base, seq_per_w)], pos_v, sp)
    for cp in idx_cps:
        cp.wait()
    gathers = [
        pltpu.async_copy(table_hbm.at[idx_v.at[c]], rows_v.at[c], gather_sems[c])
        for c in range(batch)
    ]
    pos_cp.wait()

    writebacks = []
    for pair in range(batch // 2):
        c0, c1 = 2 * pair, 2 * pair + 1
        gathers[c0].wait()
        gathers[c1].wait()

        def row(r, carry, _c0=c0, _c1=c1):
            slices = [pl.ds(cc * _LANES, _LANES) for cc in range(n_col_chunks)]
            pos_regs = [pos_v[r, sl] for sl in slices]
            for c in (_c0, _c1):
                for cc, sl in enumerate(slices):
                    rows_v[c, r, sl] = rows_v[c, r, sl] * scale + pos_regs[cc]
            return carry

        lax.fori_loop(0, seq_per_w, row, 0)
        for c in (c0, c1):
            writebacks.append(
                pltpu.async_copy(
                    rows_v.at[c],
                    out_hbm.at[c, pl.ds(seq_base, seq_per_w)], sw))
    for wb in writebacks:
        wb.wait()


def kernel(x, table, pos_encoding):
    batch, seq = x.shape
    _, d_emb = table.shape
    n = batch * seq
    seq_per_w = seq // _NUM_WORKERS
    scale = math.sqrt(d_emb)

    mesh = plsc.VectorSubcoreMesh(core_axis_name="c", subcore_axis_name="s")
    body = functools.partial(_emb_body, batch, seq, d_emb, scale)
    run = pl.kernel(
        body,
        mesh=mesh,
        out_type=jax.ShapeDtypeStruct((batch, seq, d_emb), jnp.float32),
        scratch_types=[
            pltpu.VMEM((batch, seq_per_w), jnp.int32),
            pltpu.VMEM((batch, seq_per_w, d_emb), jnp.float32),
            pltpu.VMEM((seq_per_w, d_emb), jnp.float32),
            pltpu.SemaphoreType.DMA,
            pltpu.SemaphoreType.DMA,
            pltpu.SemaphoreType.DMA,
            pltpu.SemaphoreType.DMA,
            pltpu.SemaphoreType.DMA,
            pltpu.SemaphoreType.DMA,
            pltpu.SemaphoreType.DMA,
        ],
    )
    return run(x, table, pos_encoding[:seq])


# speedup vs baseline: 1.0280x; 1.0075x over previous
import functools
import math

import jax
import jax.numpy as jnp
from jax import lax
from jax.experimental import pallas as pl
from jax.experimental.pallas import tpu as pltpu
from jax.experimental.pallas import tpu_sc as plsc

_NUM_CORES = 2
_NUM_SUBCORES = 16
_NUM_WORKERS = _NUM_CORES * _NUM_SUBCORES
_LANES = 16


def _emb_body(batch, seq, d_emb, scale,
              x_hbm, table_hbm, pos_hbm, out_hbm,
              idx_v, rows_v, pos_v, si, sp, sg0, sg1, sg2, sg3, sw):
    wid = lax.axis_index("s") * _NUM_CORES + lax.axis_index("c")
    seq_per_w = seq // _NUM_WORKERS
    seq_base = wid * seq_per_w
    n_col_chunks = d_emb // _LANES
    gather_sems = (sg0, sg1, sg2, sg3)

    idx_cps = [
        pltpu.async_copy(x_hbm.at[c, pl.ds(seq_base, seq_per_w)], idx_v.at[c], si)
        for c in range(batch)
    ]
    pos_cp = pltpu.async_copy(pos_hbm.at[pl.ds(seq_base, seq_per_w)], pos_v, sp)
    for cp in idx_cps:
        cp.wait()
    gathers = [
        pltpu.async_copy(table_hbm.at[idx_v.at[c]], rows_v.at[c], gather_sems[c])
        for c in range(batch)
    ]
    pos_cp.wait()

    writebacks = []
    for pair in range(batch // 2):
        c0, c1 = 2 * pair, 2 * pair + 1
        gathers[c0].wait()
        gathers[c1].wait()

        def row(r, carry, _c0=c0, _c1=c1):
            slices = [pl.ds(cc * _LANES, _LANES) for cc in range(n_col_chunks)]
            pos_regs = [pos_v[r, sl] for sl in slices]
            for c in (_c0, _c1):
                for cc, sl in enumerate(slices):
                    rows_v[c, r, sl] = rows_v[c, r, sl] * scale + pos_regs[cc]
            return carry

        half = seq_per_w // 2
        for h in range(2):
            lax.fori_loop(h * half, (h + 1) * half, row, 0)
            for c in (c0, c1):
                writebacks.append(
                    pltpu.async_copy(
                        rows_v.at[c, pl.ds(h * half, half)],
                        out_hbm.at[c, pl.ds(seq_base + h * half, half)], sw))
    for wb in writebacks:
        wb.wait()


def kernel(x, table, pos_encoding):
    batch, seq = x.shape
    _, d_emb = table.shape
    n = batch * seq
    seq_per_w = seq // _NUM_WORKERS
    scale = math.sqrt(d_emb)

    mesh = plsc.VectorSubcoreMesh(core_axis_name="c", subcore_axis_name="s")
    body = functools.partial(_emb_body, batch, seq, d_emb, scale)
    run = pl.kernel(
        body,
        mesh=mesh,
        out_type=jax.ShapeDtypeStruct((batch, seq, d_emb), jnp.float32),
        scratch_types=[
            pltpu.VMEM((batch, seq_per_w), jnp.int32),
            pltpu.VMEM((batch, seq_per_w, d_emb), jnp.float32),
            pltpu.VMEM((seq_per_w, d_emb), jnp.float32),
            pltpu.SemaphoreType.DMA,
            pltpu.SemaphoreType.DMA,
            pltpu.SemaphoreType.DMA,
            pltpu.SemaphoreType.DMA,
            pltpu.SemaphoreType.DMA,
            pltpu.SemaphoreType.DMA,
            pltpu.SemaphoreType.DMA,
        ],
    )
    return run(x, table, pos_encoding[:seq])


# speedup vs baseline: 1.0528x; 1.0241x over previous
import functools
import math

import jax
import jax.numpy as jnp
from jax import lax
from jax.experimental import pallas as pl
from jax.experimental.pallas import tpu as pltpu
from jax.experimental.pallas import tpu_sc as plsc

_NUM_CORES = 2
_NUM_SUBCORES = 16
_NUM_WORKERS = _NUM_CORES * _NUM_SUBCORES
_LANES = 16


def _emb_body(batch, seq, d_emb, scale,
              x_hbm, table_hbm, pos_hbm, out_hbm,
              idx_v, rows_v, pos_v,
              si, sp, sg0, sg1, sg2, sg3, sg4, sg5, sg6, sg7, sw):
    wid = lax.axis_index("s") * _NUM_CORES + lax.axis_index("c")
    seq_per_w = seq // _NUM_WORKERS
    seq_base = wid * seq_per_w
    n_col_chunks = d_emb // _LANES
    half = seq_per_w // 2
    gather_sems = ((sg0, sg1), (sg2, sg3), (sg4, sg5), (sg6, sg7))

    idx_cps = [
        pltpu.async_copy(x_hbm.at[c, pl.ds(seq_base, seq_per_w)], idx_v.at[c], si)
        for c in range(batch)
    ]
    pos_cp = pltpu.async_copy(pos_hbm.at[pl.ds(seq_base, seq_per_w)], pos_v, sp)
    for cp in idx_cps:
        cp.wait()
    gathers = [
        [
            pltpu.async_copy(
                table_hbm.at[idx_v.at[c, pl.ds(h * half, half)]],
                rows_v.at[c, pl.ds(h * half, half)], gather_sems[c][h])
            for h in range(2)
        ]
        for c in range(batch)
    ]
    pos_cp.wait()

    writebacks = []
    for pair in range(batch // 2):
        c0, c1 = 2 * pair, 2 * pair + 1

        def row(r, carry, _c0=c0, _c1=c1):
            slices = [pl.ds(cc * _LANES, _LANES) for cc in range(n_col_chunks)]
            pos_regs = [pos_v[r, sl] for sl in slices]
            for c in (_c0, _c1):
                for cc, sl in enumerate(slices):
                    rows_v[c, r, sl] = rows_v[c, r, sl] * scale + pos_regs[cc]
            return carry

        for h in range(2):
            gathers[c0][h].wait()
            gathers[c1][h].wait()
            lax.fori_loop(h * half, (h + 1) * half, row, 0)
            for c in (c0, c1):
                writebacks.append(
                    pltpu.async_copy(
                        rows_v.at[c, pl.ds(h * half, half)],
                        out_hbm.at[c, pl.ds(seq_base + h * half, half)], sw))
    for wb in writebacks:
        wb.wait()


def kernel(x, table, pos_encoding):
    batch, seq = x.shape
    _, d_emb = table.shape
    n = batch * seq
    seq_per_w = seq // _NUM_WORKERS
    scale = math.sqrt(d_emb)

    mesh = plsc.VectorSubcoreMesh(core_axis_name="c", subcore_axis_name="s")
    body = functools.partial(_emb_body, batch, seq, d_emb, scale)
    run = pl.kernel(
        body,
        mesh=mesh,
        out_type=jax.ShapeDtypeStruct((batch, seq, d_emb), jnp.float32),
        scratch_types=[
            pltpu.VMEM((batch, seq_per_w), jnp.int32),
            pltpu.VMEM((batch, seq_per_w, d_emb), jnp.float32),
            pltpu.VMEM((seq_per_w, d_emb), jnp.float32),
            pltpu.SemaphoreType.DMA,
            pltpu.SemaphoreType.DMA,
            pltpu.SemaphoreType.DMA,
            pltpu.SemaphoreType.DMA,
            pltpu.SemaphoreType.DMA,
            pltpu.SemaphoreType.DMA,
            pltpu.SemaphoreType.DMA,
            pltpu.SemaphoreType.DMA,
            pltpu.SemaphoreType.DMA,
            pltpu.SemaphoreType.DMA,
            pltpu.SemaphoreType.DMA,
        ],
    )
    return run(x, table, pos_encoding[:seq])


# speedup vs baseline: 1.0586x; 1.0055x over previous
import functools
import math

import jax
import jax.numpy as jnp
from jax import lax
from jax.experimental import pallas as pl
from jax.experimental.pallas import tpu as pltpu
from jax.experimental.pallas import tpu_sc as plsc

_NUM_CORES = 2
_NUM_SUBCORES = 16
_NUM_WORKERS = _NUM_CORES * _NUM_SUBCORES
_LANES = 16


def _emb_body(batch, seq, d_emb, scale,
              x_hbm, table_hbm, pos_hbm, out_hbm,
              idx_v, rows_v, pos_v,
              si, sp, sg0, sg1, sg2, sg3, sg4, sg5, sg6, sg7, sw):
    wid = lax.axis_index("s") * _NUM_CORES + lax.axis_index("c")
    seq_per_w = seq // _NUM_WORKERS
    seq_base = wid * seq_per_w
    n_col_chunks = d_emb // _LANES
    half = seq_per_w // 2
    gather_sems = ((sg0, sg1), (sg2, sg3), (sg4, sg5), (sg6, sg7))

    idx_cps = [
        pltpu.async_copy(x_hbm.at[c, pl.ds(seq_base, seq_per_w)], idx_v.at[c], si)
        for c in range(batch)
    ]
    for cp in idx_cps:
        cp.wait()

    def start_gather(c, h):
        return pltpu.async_copy(
            table_hbm.at[idx_v.at[c, pl.ds(h * half, half)]],
            rows_v.at[c, pl.ds(h * half, half)], gather_sems[c][h])

    gathers = [[None, None] for _ in range(batch)]
    gathers[0][0] = start_gather(0, 0)
    gathers[1][0] = start_gather(1, 0)
    pos_cp = pltpu.async_copy(pos_hbm.at[pl.ds(seq_base, seq_per_w)], pos_v, sp)
    for c, h in ((0, 1), (1, 1), (2, 0), (3, 0), (2, 1), (3, 1)):
        gathers[c][h] = start_gather(c, h)
    pos_cp.wait()

    writebacks = []
    for pair in range(batch // 2):
        c0, c1 = 2 * pair, 2 * pair + 1

        def row(r, carry, _c0=c0, _c1=c1):
            slices = [pl.ds(cc * _LANES, _LANES) for cc in range(n_col_chunks)]
            pos_regs = [pos_v[r, sl] for sl in slices]
            for c in (_c0, _c1):
                for cc, sl in enumerate(slices):
                    rows_v[c, r, sl] = rows_v[c, r, sl] * scale + pos_regs[cc]
            return carry

        for h in range(2):
            gathers[c0][h].wait()
            gathers[c1][h].wait()
            lax.fori_loop(h * half, (h + 1) * half, row, 0)
            for c in (c0, c1):
                writebacks.append(
                    pltpu.async_copy(
                        rows_v.at[c, pl.ds(h * half, half)],
                        out_hbm.at[c, pl.ds(seq_base + h * half, half)], sw))
    for wb in writebacks:
        wb.wait()


def kernel(x, table, pos_encoding):
    batch, seq = x.shape
    _, d_emb = table.shape
    n = batch * seq
    seq_per_w = seq // _NUM_WORKERS
    scale = math.sqrt(d_emb)

    mesh = plsc.VectorSubcoreMesh(core_axis_name="c", subcore_axis_name="s")
    body = functools.partial(_emb_body, batch, seq, d_emb, scale)
    run = pl.kernel(
        body,
        mesh=mesh,
        out_type=jax.ShapeDtypeStruct((batch, seq, d_emb), jnp.float32),
        scratch_types=[
            pltpu.VMEM((batch, seq_per_w), jnp.int32),
            pltpu.VMEM((batch, seq_per_w, d_emb), jnp.float32),
            pltpu.VMEM((seq_per_w, d_emb), jnp.float32),
            pltpu.SemaphoreType.DMA,
            pltpu.SemaphoreType.DMA,
            pltpu.SemaphoreType.DMA,
            pltpu.SemaphoreType.DMA,
            pltpu.SemaphoreType.DMA,
            pltpu.SemaphoreType.DMA,
            pltpu.SemaphoreType.DMA,
            pltpu.SemaphoreType.DMA,
            pltpu.SemaphoreType.DMA,
            pltpu.SemaphoreType.DMA,
            pltpu.SemaphoreType.DMA,
        ],
    )
    return run(x, table, pos_encoding[:seq])


# speedup vs baseline: 1.0599x; 1.0013x over previous
import functools
import math

import jax
import jax.numpy as jnp
from jax import lax
from jax.experimental import pallas as pl
from jax.experimental.pallas import tpu as pltpu
from jax.experimental.pallas import tpu_sc as plsc

_NUM_CORES = 2
_NUM_SUBCORES = 16
_NUM_WORKERS = _NUM_CORES * _NUM_SUBCORES
_LANES = 16


def _emb_body(batch, seq, d_emb, scale,
              x_hbm, table_hbm, pos_hbm, out_hbm,
              idx_v, rows_v, pos_v,
              si, sp, sg0, sg1, sg2, sg3, sg4, sg5, sg6, sg7, sw):
    wid = lax.axis_index("s") * _NUM_CORES + lax.axis_index("c")
    seq_per_w = seq // _NUM_WORKERS
    seq_base = wid * seq_per_w
    n_col_chunks = d_emb // _LANES
    half = seq_per_w // 2
    gather_sems = ((sg0, sg1), (sg2, sg3), (sg4, sg5), (sg6, sg7))

    idx_cps = [
        pltpu.async_copy(x_hbm.at[c, pl.ds(seq_base, seq_per_w)], idx_v.at[c], si)
        for c in range(batch)
    ]
    for cp in idx_cps:
        cp.wait()

    def start_gather(c, h):
        return pltpu.async_copy(
            table_hbm.at[idx_v.at[c, pl.ds(h * half, half)]],
            rows_v.at[c, pl.ds(h * half, half)], gather_sems[c][h])

    gathers = [[None, None] for _ in range(batch)]
    gathers[0][0] = start_gather(0, 0)
    gathers[1][0] = start_gather(1, 0)
    pos_cp = pltpu.async_copy(pos_hbm.at[pl.ds(seq_base, seq_per_w)], pos_v, sp)
    for c, h in ((0, 1), (1, 1), (2, 0), (3, 0), (2, 1), (3, 1)):
        gathers[c][h] = start_gather(c, h)
    pos_cp.wait()

    writebacks = []
    for pair in range(batch // 2):
        c0, c1 = 2 * pair, 2 * pair + 1

        def row(r, carry, _c0=c0, _c1=c1):
            slices = [pl.ds(cc * _LANES, _LANES) for cc in range(n_col_chunks)]
            pos_regs = [pos_v[r, sl] for sl in slices]
            for c in (_c0, _c1):
                for cc, sl in enumerate(slices):
                    rows_v[c, r, sl] = rows_v[c, r, sl] * scale + pos_regs[cc]
            return carry

        for h in range(2):
            gathers[c0][h].wait()
            gathers[c1][h].wait()
            lax.fori_loop(h * half, (h + 1) * half, row, 0)
            for c in (c0, c1):
                writebacks.append(
                    pltpu.async_copy(
                        rows_v.at[c, pl.ds(h * half, half)],
                        out_hbm.at[c, pl.ds(seq_base + h * half, half)], sw))
    for wb in writebacks:
        wb.wait()


def kernel(x, table, pos_encoding):
    batch, seq = x.shape
    _, d_emb = table.shape
    seq_per_w = seq // _NUM_WORKERS
    scale = math.sqrt(d_emb)

    mesh = plsc.VectorSubcoreMesh(core_axis_name="c", subcore_axis_name="s")
    body = functools.partial(_emb_body, batch, seq, d_emb, scale)
    run = pl.kernel(
        body,
        mesh=mesh,
        out_type=jax.ShapeDtypeStruct((batch, seq, d_emb), jnp.float32),
        scratch_types=[
            pltpu.VMEM((batch, seq_per_w), jnp.int32),
            pltpu.VMEM((batch, seq_per_w, d_emb), jnp.float32),
            pltpu.VMEM((seq_per_w, d_emb), jnp.float32),
            pltpu.SemaphoreType.DMA,
            pltpu.SemaphoreType.DMA,
            pltpu.SemaphoreType.DMA,
            pltpu.SemaphoreType.DMA,
            pltpu.SemaphoreType.DMA,
            pltpu.SemaphoreType.DMA,
            pltpu.SemaphoreType.DMA,
            pltpu.SemaphoreType.DMA,
            pltpu.SemaphoreType.DMA,
            pltpu.SemaphoreType.DMA,
            pltpu.SemaphoreType.DMA,
        ],
    )
    return run(x, table, pos_encoding[:seq])
